# baseline (speedup 1.0000x reference)
import functools

import jax
import jax.numpy as jnp
from jax import lax
from jax.experimental import pallas as pl
from jax.experimental.pallas import tpu as pltpu


def _moe_body(E, x_ref, gw_ref, wg_ref, wu_ref, wd_ref, out_ref, cw_ref):
    e = pl.program_id(1)
    x = x_ref[...]

    @pl.when(e == 0)
    def _router():
        logits = lax.dot_general(
            x, gw_ref[...], (((1,), (1,)), ((), ())),
            preferred_element_type=jnp.float32)
        iota = lax.broadcasted_iota(jnp.int32, logits.shape, 1)
        m1 = jnp.max(logits, axis=1, keepdims=True)
        a1 = jnp.min(jnp.where(logits == m1, iota, E), axis=1, keepdims=True)
        masked = jnp.where(iota == a1, -jnp.inf, logits)
        m2 = jnp.max(masked, axis=1, keepdims=True)
        a2 = jnp.min(jnp.where(masked == m2, iota, E), axis=1, keepdims=True)
        p1 = 1.0 / (1.0 + jnp.exp(m2 - m1))
        p2 = 1.0 - p1
        cw_ref[...] = jnp.where(iota == a1, p1, 0.0) + jnp.where(iota == a2, p2, 0.0)

    wg = wg_ref[0]
    wu = wu_ref[0]
    wd = wd_ref[0]
    gate_h = lax.dot_general(x, wg, (((1,), (1,)), ((), ())),
                             preferred_element_type=jnp.float32)
    up_h = lax.dot_general(x, wu, (((1,), (1,)), ((), ())),
                           preferred_element_type=jnp.float32)
    h = gate_h * jax.nn.sigmoid(gate_h) * up_h
    y = lax.dot_general(h, wd, (((1,), (1,)), ((), ())),
                        preferred_element_type=jnp.float32)
    cw = cw_ref[...]
    lane = lax.broadcasted_iota(jnp.int32, cw.shape, 1)
    w_e = jnp.sum(jnp.where(lane == e, cw, 0.0), axis=1, keepdims=True)
    contrib = w_e * y

    @pl.when(e == 0)
    def _init():
        out_ref[...] = contrib

    @pl.when(e > 0)
    def _acc():
        out_ref[...] += contrib


def kernel(hidden_states, gate_w, w_gate, w_up, w_down):
    orig_shape = hidden_states.shape
    D = orig_shape[-1]
    x = hidden_states.reshape(-1, D)
    T = x.shape[0]
    E, DFF = w_gate.shape[0], w_gate.shape[1]
    BT = min(256, T)
    NT = T // BT

    out = pl.pallas_call(
        functools.partial(_moe_body, E),
        grid=(NT, E),
        in_specs=[
            pl.BlockSpec((BT, D), lambda t, e: (t, 0)),
            pl.BlockSpec((E, D), lambda t, e: (0, 0)),
            pl.BlockSpec((1, DFF, D), lambda t, e: (e, 0, 0)),
            pl.BlockSpec((1, DFF, D), lambda t, e: (e, 0, 0)),
            pl.BlockSpec((1, D, DFF), lambda t, e: (e, 0, 0)),
        ],
        out_specs=pl.BlockSpec((BT, D), lambda t, e: (t, 0)),
        out_shape=jax.ShapeDtypeStruct((T, D), jnp.float32),
        scratch_shapes=[pltpu.VMEM((BT, E), jnp.float32)],
        compiler_params=pltpu.CompilerParams(
            dimension_semantics=("arbitrary", "arbitrary")),
    )(x, gate_w, w_gate, w_up, w_down)
    return out.reshape(orig_shape)


# speedup vs baseline: 1.9322x; 1.9322x over previous
import functools

import jax
import jax.numpy as jnp
from jax import lax
from jax.experimental import pallas as pl
from jax.experimental.pallas import tpu as pltpu
from jax.experimental.pallas import tpu_sc as plsc

NW = 32
CT = 128
BTF = 256
L = 16


def _router_body(E, x_ref, gw_ref, a1_ref, a2_ref, p1_ref, p2_ref, cnt_ref):
    x = x_ref[...]
    logits = lax.dot_general(x, gw_ref[...], (((1,), (1,)), ((), ())),
                             preferred_element_type=jnp.float32)
    iota = lax.broadcasted_iota(jnp.int32, logits.shape, 1)
    m1 = jnp.max(logits, axis=1, keepdims=True)
    a1 = jnp.min(jnp.where(logits == m1, iota, E), axis=1, keepdims=True)
    masked = jnp.where(iota == a1, -jnp.inf, logits)
    m2 = jnp.max(masked, axis=1, keepdims=True)
    a2 = jnp.min(jnp.where(masked == m2, iota, E), axis=1, keepdims=True)
    p1 = 1.0 / (1.0 + jnp.exp(m2 - m1))
    a1_ref[...] = a1
    a2_ref[...] = a2
    p1_ref[...] = p1
    p2_ref[...] = 1.0 - p1
    BT = x.shape[0]
    lane16 = lax.broadcasted_iota(jnp.int32, (BT, L), 1)
    onehot = ((lane16 == a1) | (lane16 == a2)).astype(jnp.float32)
    nch = BT // CT
    sel_r = lax.broadcasted_iota(jnp.int32, (nch, BT), 0)
    sel_t = lax.broadcasted_iota(jnp.int32, (nch, BT), 1) // CT
    sel = (sel_r == sel_t).astype(jnp.float32)
    cnt = lax.dot_general(sel, onehot, (((1,), (0,)), ((), ())),
                          preferred_element_type=jnp.float32
                          ).astype(jnp.int32)
    cnt_ref[...] = cnt.reshape(1, nch, L)


def _dispatch_body(T, PAD, a1_hbm, a2_hbm, cnt_hbm, x_hbm,
                   xs_hbm, inv0_hbm, inv1_hbm, blk_hbm,
                   cnt_vm, a_vm, pos_stage, pos_sc, cursor_vm, blk_vm,
                   xrows, sem):
    wid = lax.axis_index("s") * 2 + lax.axis_index("c")
    base = wid * CT
    lane = lax.iota(jnp.int32, L)

    pltpu.sync_copy(cnt_hbm, cnt_vm)
    pltpu.sync_copy(a1_hbm.at[pl.ds(base, CT)], a_vm.at[0])
    pltpu.sync_copy(a2_hbm.at[pl.ds(base, CT)], a_vm.at[1])

    tot = jnp.zeros((L,), jnp.int32)
    pre = jnp.zeros((L,), jnp.int32)
    for w in range(NW):
        c = cnt_vm[w, :]
        tot = tot + c
        pre = pre + jnp.where(w < wid, c, 0)
    ru = (tot + (BTF - 1)) & ~(BTF - 1)
    ao = jnp.cumsum(ru) - ru
    cursor_vm[...] = ao + pre

    @pl.when(wid == 0)
    def _blocks():
        nvec = blk_vm.shape[0] // L
        bstart = ao >> 8 if BTF == 256 else ao // BTF
        bend = (ao + ru) // BTF
        for v in range(nvec):
            g = lane + v * L
            be = jnp.full((L,), -1, jnp.int32)
            for e in range(8):
                s_e = lax.reduce_sum(jnp.where(lane == e, bstart, 0), (0,))
                t_e = lax.reduce_sum(jnp.where(lane == e, bend, 0), (0,))
                be = jnp.where((g >= s_e) & (g < t_e), e, be)
            blk_vm[pl.ds(v * L, L)] = be
        pltpu.sync_copy(blk_vm, blk_hbm)

    for k in range(2):
        for j in range(CT // L):
            eid = a_vm[k, pl.ds(j * L, L)]
            cur = cursor_vm[...]
            rank = jnp.zeros((L,), jnp.int32)
            inc = jnp.zeros((L,), jnp.int32)
            for e in range(8):
                m = eid == e
                mi = m.astype(jnp.int32)
                rank = rank + jnp.where(m, jnp.cumsum(mi) - mi, 0)
                pc = plsc.all_reduce_population_count(m)
                inc = inc + jnp.where(lane == e, pc, 0)
            pos = plsc.load_gather(cursor_vm, [eid]) + rank
            cursor_vm[...] = cur + inc
            pos_stage[k, pl.ds(j * L, L)] = pos
            r = k * 2 + j // 4
            pos_sc[r, pl.ds((j % 4) * L, L)] = pos

    pltpu.sync_copy(pos_stage.at[0], inv0_hbm.at[pl.ds(base, CT)])
    pltpu.sync_copy(pos_stage.at[1], inv1_hbm.at[pl.ds(base, CT)])

    for h in range(2):
        pltpu.sync_copy(x_hbm.at[pl.ds(base + h * 64, 64)], xrows)
        for k in range(2):
            pltpu.async_copy(xrows, xs_hbm.at[pos_sc.at[k * 2 + h]],
                             sem).wait()


def _ffn_body(be_ref, xs_ref, wg_ref, wu_ref, wd_ref, ys_ref):
    i = pl.program_id(0)
    be = be_ref[i]

    @pl.when(be >= 0)
    def _compute():
        x = xs_ref[...]
        wg = wg_ref[0]
        wu = wu_ref[0]
        wd = wd_ref[0]
        gate_h = lax.dot_general(x, wg, (((1,), (1,)), ((), ())),
                                 preferred_element_type=jnp.float32)
        up_h = lax.dot_general(x, wu, (((1,), (1,)), ((), ())),
                               preferred_element_type=jnp.float32)
        h = gate_h * jax.nn.sigmoid(gate_h) * up_h
        ys_ref[...] = lax.dot_general(h, wd, (((1,), (1,)), ((), ())),
                                      preferred_element_type=jnp.float32)


def _combine_body(T, D, ys_hbm, inv0_hbm, inv1_hbm, p1_hbm, p2_hbm,
                  out_hbm, idx_lin, idx_rows, p_vm, rowsA, rowsB, rowsC, sem):
    wid = lax.axis_index("s") * 2 + lax.axis_index("c")
    base = wid * CT
    lane = lax.iota(jnp.int32, L)
    RG = rowsA.shape[0]
    nrounds = CT // RG
    nv = D // L

    pltpu.sync_copy(inv0_hbm.at[pl.ds(base, CT)], idx_lin.at[0])
    pltpu.sync_copy(inv1_hbm.at[pl.ds(base, CT)], idx_lin.at[1])
    pltpu.sync_copy(p1_hbm.at[pl.ds(base, CT)], p_vm.at[0])
    pltpu.sync_copy(p2_hbm.at[pl.ds(base, CT)], p_vm.at[1])
    for k in range(2):
        for r in range(nrounds):
            for j in range(RG // L):
                v = idx_lin[k, pl.ds(r * RG + j * L, L)]
                idx_rows[k * nrounds + r, pl.ds(j * L, L)] = v

    for r in range(nrounds):
        pltpu.async_copy(ys_hbm.at[idx_rows.at[r]], rowsA, sem).wait()
        pltpu.async_copy(ys_hbm.at[idx_rows.at[nrounds + r]], rowsB,
                         sem).wait()
        for g in range(RG // L):
            w0v = p_vm[0, pl.ds(r * RG + g * L, L)]
            w1v = p_vm[1, pl.ds(r * RG + g * L, L)]

            def row_fn(i2, _, w0v=w0v, w1v=w1v, g=g):
                w0 = lax.reduce_sum(jnp.where(lane == i2, w0v, 0.0), (0,))
                w1 = lax.reduce_sum(jnp.where(lane == i2, w1v, 0.0), (0,))
                i = g * L + i2
                for d in range(nv):
                    rowsC[i, pl.ds(d * L, L)] = (
                        w0 * rowsA[i, pl.ds(d * L, L)]
                        + w1 * rowsB[i, pl.ds(d * L, L)])
                return 0

            lax.fori_loop(0, L, row_fn, 0)
        pltpu.sync_copy(rowsC, out_hbm.at[pl.ds(base + r * RG, RG)])


def kernel(hidden_states, gate_w, w_gate, w_up, w_down):
    orig_shape = hidden_states.shape
    D = orig_shape[-1]
    x = hidden_states.reshape(-1, D)
    T = x.shape[0]
    E, DFF = w_gate.shape[0], w_gate.shape[1]
    PAD = T * 2 + E * BTF
    NBLK = PAD // BTF
    BT = 512
    NT = T // BT

    a1, a2, p1, p2, cnt = pl.pallas_call(
        functools.partial(_router_body, E),
        grid=(NT,),
        in_specs=[
            pl.BlockSpec((BT, D), lambda t: (t, 0)),
            pl.BlockSpec((E, D), lambda t: (0, 0)),
        ],
        out_specs=[
            pl.BlockSpec((BT, 1), lambda t: (t, 0)),
            pl.BlockSpec((BT, 1), lambda t: (t, 0)),
            pl.BlockSpec((BT, 1), lambda t: (t, 0)),
            pl.BlockSpec((BT, 1), lambda t: (t, 0)),
            pl.BlockSpec((1, BT // CT, L), lambda t: (t, 0, 0)),
        ],
        out_shape=[
            jax.ShapeDtypeStruct((T, 1), jnp.int32),
            jax.ShapeDtypeStruct((T, 1), jnp.int32),
            jax.ShapeDtypeStruct((T, 1), jnp.float32),
            jax.ShapeDtypeStruct((T, 1), jnp.float32),
            jax.ShapeDtypeStruct((NT, BT // CT, L), jnp.int32),
        ],
        compiler_params=pltpu.CompilerParams(
            dimension_semantics=("arbitrary",)),
    )(x, gate_w)

    a1 = a1.reshape(T)
    a2 = a2.reshape(T)
    p1 = p1.reshape(T)
    p2 = p2.reshape(T)
    cnt = cnt.reshape(NW, L)

    mesh = plsc.VectorSubcoreMesh(core_axis_name="c", subcore_axis_name="s")
    NBV = ((NBLK + L - 1) // L) * L

    dispatch = pl.kernel(
        functools.partial(_dispatch_body, T, PAD),
        mesh=mesh,
        out_type=[
            jax.ShapeDtypeStruct((PAD, D), jnp.float32),
            jax.ShapeDtypeStruct((T,), jnp.int32),
            jax.ShapeDtypeStruct((T,), jnp.int32),
            jax.ShapeDtypeStruct((NBV,), jnp.int32),
        ],
        scratch_types=[
            pltpu.VMEM((NW, L), jnp.int32),
            pltpu.VMEM((2, CT), jnp.int32),
            pltpu.VMEM((2, CT), jnp.int32),
            pltpu.VMEM((4, 64), jnp.int32),
            pltpu.VMEM((L,), jnp.int32),
            pltpu.VMEM((NBV,), jnp.int32),
            pltpu.VMEM((64, D), jnp.float32),
            pltpu.SemaphoreType.DMA,
        ],
        compiler_params=pltpu.CompilerParams(needs_layout_passes=False),
    )
    xs, inv0, inv1, blk = dispatch(a1, a2, cnt, x)

    grid_spec = pltpu.PrefetchScalarGridSpec(
        num_scalar_prefetch=1,
        grid=(NBLK,),
        in_specs=[
            pl.BlockSpec((BTF, D), lambda i, be: (i, 0)),
            pl.BlockSpec((1, DFF, D), lambda i, be: (jnp.maximum(be[i], 0), 0, 0)),
            pl.BlockSpec((1, DFF, D), lambda i, be: (jnp.maximum(be[i], 0), 0, 0)),
            pl.BlockSpec((1, D, DFF), lambda i, be: (jnp.maximum(be[i], 0), 0, 0)),
        ],
        out_specs=pl.BlockSpec((BTF, D), lambda i, be: (i, 0)),
    )
    ys = pl.pallas_call(
        _ffn_body,
        grid_spec=grid_spec,
        out_shape=jax.ShapeDtypeStruct((PAD, D), jnp.float32),
        compiler_params=pltpu.CompilerParams(
            dimension_semantics=("arbitrary",)),
    )(blk, xs, w_gate, w_up, w_down)

    combine = pl.kernel(
        functools.partial(_combine_body, T, D),
        mesh=mesh,
        out_type=jax.ShapeDtypeStruct((T, D), jnp.float32),
        scratch_types=[
            pltpu.VMEM((2, CT), jnp.int32),
            pltpu.VMEM((2 * (CT // 32), 32), jnp.int32),
            pltpu.VMEM((2, CT), jnp.float32),
            pltpu.VMEM((32, D), jnp.float32),
            pltpu.VMEM((32, D), jnp.float32),
            pltpu.VMEM((32, D), jnp.float32),
            pltpu.SemaphoreType.DMA,
        ],
        compiler_params=pltpu.CompilerParams(needs_layout_passes=False),
    )
    out = combine(ys, inv0, inv1, p1, p2)
    return out.reshape(orig_shape)
